# baseline (speedup 1.0000x reference)
import functools

import jax
import jax.numpy as jnp
from jax import lax
from jax.experimental import pallas as pl
from jax.experimental.pallas import tpu as pltpu
from jax.experimental.pallas import tpu_sc as plsc

_BATCH = 16384
_D = 128
_LANES = 16
_NW = 32
_BPW = _BATCH // _NW
_CH = 64
_NCH = _BPW // _CH

_mesh = plsc.VectorSubcoreMesh(core_axis_name="c", subcore_axis_name="s")


@functools.partial(
    pl.kernel,
    out_type=jax.ShapeDtypeStruct((_BATCH,), jnp.float32),
    mesh=_mesh,
    scratch_types=[
        pltpu.VMEM((_BPW,), jnp.int32),
        pltpu.VMEM((_BPW,), jnp.int32),
        pltpu.VMEM((3 * _CH, _D), jnp.float32),
        pltpu.VMEM((3 * _CH, _D), jnp.float32),
        pltpu.VMEM((_BPW,), jnp.float32),
        pltpu.VMEM((_BPW,), jnp.float32),
        pltpu.VMEM((_BPW,), jnp.float32),
        pltpu.VMEM((1,), jnp.float32),
        pltpu.VMEM((_LANES * _LANES,), jnp.float32),
        pltpu.SemaphoreType.DMA((3,)),
        pltpu.SemaphoreType.DMA,
    ],
    compiler_params=pltpu.CompilerParams(needs_layout_passes=False),
)
def _svd_sc(uid_hbm, iid_hbm, ut_hbm, it_hbm, ub_hbm, ib_hbm, gb_hbm,
            out_hbm, idx_u, idx_i, u_rows, v_rows, outv, bu, bi, gv,
            prt, sems, sem_b):
    wid = lax.axis_index("s") * 2 + lax.axis_index("c")
    base = wid * _BPW

    iu0 = pltpu.make_async_copy(
        uid_hbm.at[pl.ds(base, _CH)], idx_u.at[pl.ds(0, _CH)], sem_b)
    ii0 = pltpu.make_async_copy(
        iid_hbm.at[pl.ds(base, _CH)], idx_i.at[pl.ds(0, _CH)], sem_b)
    iu0.start()
    ii0.start()
    iu1 = pltpu.make_async_copy(
        uid_hbm.at[pl.ds(base + _CH, _BPW - _CH)],
        idx_u.at[pl.ds(_CH, _BPW - _CH)], sem_b)
    ii1 = pltpu.make_async_copy(
        iid_hbm.at[pl.ds(base + _CH, _BPW - _CH)],
        idx_i.at[pl.ds(_CH, _BPW - _CH)], sem_b)
    gv_cp = pltpu.make_async_copy(gb_hbm, gv, sem_b)
    iu0.wait()
    ii0.wait()

    def chunk_copies(c, par):
        off = par * _CH
        u_cp = pltpu.make_async_copy(
            ut_hbm.at[idx_u.at[pl.ds(c * _CH, _CH)]],
            u_rows.at[pl.ds(off, _CH)], sems.at[par])
        v_cp = pltpu.make_async_copy(
            it_hbm.at[idx_i.at[pl.ds(c * _CH, _CH)]],
            v_rows.at[pl.ds(off, _CH)], sems.at[par])
        return u_cp, v_cp

    def start_chunk(c, par):
        u_cp, v_cp = chunk_copies(c, par)
        u_cp.start()
        v_cp.start()

    start_chunk(0, 0)

    iu1.start()
    ii1.start()
    gv_cp.start()
    iu1.wait()
    ii1.wait()

    start_chunk(1, 1)

    bu_cp = pltpu.make_async_copy(ub_hbm.at[idx_u], bu, sem_b)
    bi_cp = pltpu.make_async_copy(ib_hbm.at[idx_i], bi, sem_b)
    bu_cp.start()
    bi_cp.start()

    def chunk_body(c, _):
        par = lax.rem(c, 3)

        @pl.when(c + 2 < _NCH)
        def _():
            start_chunk(c + 2, lax.rem(c + 2, 3))

        u_cp, v_cp = chunk_copies(c, par)
        u_cp.wait()
        v_cp.wait()

        roff = par * _CH

        def group_body(g, _):
            for e in range(_LANES):
                i = roff + g * _LANES + e
                p = [u_rows[i, pl.ds(j * _LANES, _LANES)]
                     * v_rows[i, pl.ds(j * _LANES, _LANES)]
                     for j in range(_D // _LANES)]
                while len(p) > 1:
                    p = [p[k] + p[k + 1] for k in range(0, len(p), 2)]
                prt[pl.ds(e * _LANES, _LANES)] = p[0]
            rows = lax.iota(jnp.int32, _LANES) * _LANES
            gs = [plsc.load_gather(prt, [rows + l]) for l in range(_LANES)]
            while len(gs) > 1:
                gs = [gs[k] + gs[k + 1] for k in range(0, len(gs), 2)]
            outv[pl.ds(c * _CH + g * _LANES, _LANES)] = gs[0]
            return 0

        lax.fori_loop(0, _CH // _LANES, group_body, 0)
        return 0

    lax.fori_loop(0, _NCH, chunk_body, 0)

    gv_cp.wait()
    bu_cp.wait()
    bi_cp.wait()

    g = plsc.load_gather(gv, [jnp.zeros((_LANES,), jnp.int32)])

    def add_body(k, _):
        sl = pl.ds(k * _LANES, _LANES)
        outv[sl] = outv[sl] + bu[sl] + bi[sl] + g
        return 0

    lax.fori_loop(0, _BPW // _LANES, add_body, 0)

    pltpu.sync_copy(outv, out_hbm.at[pl.ds(base, _BPW)])


def kernel(user_ids, item_ids, user_table, item_table, user_bias,
           item_bias, global_bias):
    uid = user_ids.astype(jnp.int32)
    iid = item_ids.astype(jnp.int32)
    gb = jnp.asarray(global_bias, jnp.float32).reshape((1,))
    return _svd_sc(uid, iid, user_table, item_table, user_bias, item_bias, gb)


# speedup vs baseline: 1.0215x; 1.0215x over previous
import functools

import jax
import jax.numpy as jnp
from jax import lax
from jax.experimental import pallas as pl
from jax.experimental.pallas import tpu as pltpu
from jax.experimental.pallas import tpu_sc as plsc

_BATCH = 16384
_D = 128
_LANES = 16
_NW = 32
_BPW = _BATCH // _NW
_CH = 64
_NCH = _BPW // _CH

_mesh = plsc.VectorSubcoreMesh(core_axis_name="c", subcore_axis_name="s")


@functools.partial(
    pl.kernel,
    out_type=jax.ShapeDtypeStruct((_BATCH,), jnp.float32),
    mesh=_mesh,
    scratch_types=[
        pltpu.VMEM((_BPW,), jnp.int32),
        pltpu.VMEM((_BPW,), jnp.int32),
        pltpu.VMEM((3 * _CH, _D), jnp.float32),
        pltpu.VMEM((3 * _CH, _D), jnp.float32),
        pltpu.VMEM((_BPW,), jnp.float32),
        pltpu.VMEM((_BPW,), jnp.float32),
        pltpu.VMEM((_BPW,), jnp.float32),
        pltpu.VMEM((_LANES,), jnp.float32),
        pltpu.VMEM((_LANES * _LANES,), jnp.float32),
        pltpu.SemaphoreType.DMA((3,)),
        pltpu.SemaphoreType.DMA,
    ],
    compiler_params=pltpu.CompilerParams(needs_layout_passes=False),
)
def _svd_sc(uid_hbm, iid_hbm, ut_hbm, it_hbm, ub_hbm, ib_hbm, gb_hbm,
            out_hbm, idx_u, idx_i, u_rows, v_rows, outv, bu, bi, gv,
            prt, sems, sem_b):
    wid = lax.axis_index("s") * 2 + lax.axis_index("c")
    base = wid * _BPW

    iu0 = pltpu.make_async_copy(
        uid_hbm.at[pl.ds(base, _CH)], idx_u.at[pl.ds(0, _CH)], sem_b)
    ii0 = pltpu.make_async_copy(
        iid_hbm.at[pl.ds(base, _CH)], idx_i.at[pl.ds(0, _CH)], sem_b)
    iu0.start()
    ii0.start()
    iu1 = pltpu.make_async_copy(
        uid_hbm.at[pl.ds(base + _CH, _BPW - _CH)],
        idx_u.at[pl.ds(_CH, _BPW - _CH)], sem_b)
    ii1 = pltpu.make_async_copy(
        iid_hbm.at[pl.ds(base + _CH, _BPW - _CH)],
        idx_i.at[pl.ds(_CH, _BPW - _CH)], sem_b)
    gv_cp = pltpu.make_async_copy(gb_hbm, gv, sem_b)
    iu0.wait()
    ii0.wait()

    def chunk_copies(c, par):
        off = par * _CH
        u_cp = pltpu.make_async_copy(
            ut_hbm.at[idx_u.at[pl.ds(c * _CH, _CH)]],
            u_rows.at[pl.ds(off, _CH)], sems.at[par])
        v_cp = pltpu.make_async_copy(
            it_hbm.at[idx_i.at[pl.ds(c * _CH, _CH)]],
            v_rows.at[pl.ds(off, _CH)], sems.at[par])
        return u_cp, v_cp

    def start_chunk(c, par):
        u_cp, v_cp = chunk_copies(c, par)
        u_cp.start()
        v_cp.start()

    start_chunk(0, 0)

    iu1.start()
    ii1.start()
    gv_cp.start()
    iu1.wait()
    ii1.wait()

    start_chunk(1, 1)

    bu_cp = pltpu.make_async_copy(ub_hbm.at[idx_u], bu, sem_b)
    bi_cp = pltpu.make_async_copy(ib_hbm.at[idx_i], bi, sem_b)
    bu_cp.start()
    bi_cp.start()

    def chunk_body(c, _):
        par = lax.rem(c, 3)

        @pl.when(c + 2 < _NCH)
        def _():
            start_chunk(c + 2, lax.rem(c + 2, 3))

        u_cp, v_cp = chunk_copies(c, par)
        u_cp.wait()
        v_cp.wait()

        roff = par * _CH

        def group_body(g, _):
            for e in range(_LANES):
                i = roff + g * _LANES + e
                p = [u_rows[i, pl.ds(j * _LANES, _LANES)]
                     * v_rows[i, pl.ds(j * _LANES, _LANES)]
                     for j in range(_D // _LANES)]
                while len(p) > 1:
                    p = [p[k] + p[k + 1] for k in range(0, len(p), 2)]
                prt[pl.ds(e * _LANES, _LANES)] = p[0]
            rows = lax.iota(jnp.int32, _LANES) * _LANES
            gs = [plsc.load_gather(prt, [rows + l]) for l in range(_LANES)]
            while len(gs) > 1:
                gs = [gs[k] + gs[k + 1] for k in range(0, len(gs), 2)]
            outv[pl.ds(c * _CH + g * _LANES, _LANES)] = gs[0]
            return 0

        lax.fori_loop(0, _CH // _LANES, group_body, 0)
        return 0

    lax.fori_loop(0, _NCH, chunk_body, 0)

    gv_cp.wait()
    bu_cp.wait()
    bi_cp.wait()

    g = gv[...]

    def add_body(k, _):
        sl = pl.ds(k * _LANES, _LANES)
        outv[sl] = outv[sl] + bu[sl] + bi[sl] + g
        return 0

    lax.fori_loop(0, _BPW // _LANES, add_body, 0)

    pltpu.sync_copy(outv, out_hbm.at[pl.ds(base, _BPW)])


def kernel(user_ids, item_ids, user_table, item_table, user_bias,
           item_bias, global_bias):
    uid = user_ids.astype(jnp.int32)
    iid = item_ids.astype(jnp.int32)
    gb = jnp.full((_LANES,), global_bias, jnp.float32)
    return _svd_sc(uid, iid, user_table, item_table, user_bias, item_bias, gb)
